# baseline (speedup 1.0000x reference)
import jax
import jax.numpy as jnp
from jax import lax
from jax.experimental import pallas as pl
from jax.experimental.pallas import tpu as pltpu
from jax.experimental.pallas import tpu_sc as plsc

B, M, N = 4, 4096, 4096
NNZ = 167772
NBLK = (NNZ + 127) // 128
LAST_VALID = NNZ - (NBLK - 1) * 128
BPT = 82
G = 8
NCH = (BPT + G - 1) // G
NBUF = 4
CW = G * 128
MN = M + N
W_PRIMAL, W_DUAL, W_STAT, W_COMP = 0.1, 0.1, 0.6, 0.2
INV_MB = 1.0 / float(M * B)


def _sc_body(x_hbm, lam_hbm, vals_hbm, rows_hbm, cols_hbm, b_hbm, c_hbm,
             out_hbm,
             vals_ch, rows_ch, cols_ch, x2_v, lam2_v, acc_v, tmp_v, bc_v,
             out_v,
             acc_sh,
             sem_v, sem_r, sem_c, sem_x, sem_l):
    c = lax.axis_index("c")
    s = lax.axis_index("s")
    base = jnp.minimum(s * BPT, NBLK - G * NCH)
    lanes = lax.iota(jnp.int32, 16)
    zero16 = jnp.zeros((16,), jnp.float32)
    zero16i = jnp.zeros((16,), jnp.int32)

    def start(ch, slot):
        @pl.when(ch < NCH)
        def _():
            col0 = (base + ch * G) * 128
            for hbm, buf, sem in ((vals_hbm, vals_ch, sem_v),
                                  (rows_hbm, rows_ch, sem_r),
                                  (cols_hbm, cols_ch, sem_c)):
                pltpu.make_async_copy(
                    hbm.at[pl.ds(0, 4), pl.ds(col0, CW)],
                    buf.at[slot], sem.at[slot]).start()

    def wait(slot):
        for hbm, buf, sem in ((vals_hbm, vals_ch, sem_v),
                              (rows_hbm, rows_ch, sem_r),
                              (cols_hbm, cols_ch, sem_c)):
            pltpu.make_async_copy(
                hbm.at[pl.ds(0, 4), pl.ds(0, CW)],
                buf.at[slot], sem.at[slot]).wait()

    def sanitize(ch, slot):
        @pl.when(jnp.logical_and(s == 15, ch == 0))
        def _():
            nover = 16 * BPT - NBLK + (G * NCH - BPT)
            for r in range(4):
                for o in range(nover * 8):
                    vals_ch[slot, r, pl.ds(o * 16, 16)] = zero16

        @pl.when(jnp.logical_and(s < 15, ch == NCH - 1))
        def _():
            for r in range(4):
                for o in range((BPT - G * (NCH - 1)) * 8, G * 8):
                    vals_ch[slot, r, pl.ds(o * 16, 16)] = zero16

        @pl.when(jnp.logical_and(s == 15, ch == NCH - 1))
        def _():
            fo = (G - 1) * 128 + (LAST_VALID // 16) * 16
            keep = lanes < (LAST_VALID - (LAST_VALID // 16) * 16)
            for r in range(4):
                vals_ch[slot, r, pl.ds(fo, 16)] = jnp.where(
                    keep, vals_ch[slot, r, pl.ds(fo, 16)], 0.0)
                rows_ch[slot, r, pl.ds(fo, 16)] = jnp.where(
                    keep, rows_ch[slot, r, pl.ds(fo, 16)], 0)
                cols_ch[slot, r, pl.ds(fo, 16)] = jnp.where(
                    keep, cols_ch[slot, r, pl.ds(fo, 16)], 0)
                for o in range(fo + 16, CW, 16):
                    vals_ch[slot, r, pl.ds(o, 16)] = zero16
                    rows_ch[slot, r, pl.ds(o, 16)] = zero16i
                    cols_ch[slot, r, pl.ds(o, 16)] = zero16i

    def process_chunk(slot):
        for pair in (0, 1):
            crow = 2 * c + pair
            xp = x2_v.at[pl.ds(pair * N, N)]
            lamp = lam2_v.at[pl.ds(pair * M, M)]
            accp = acc_v.at[pl.ds(pair * MN, MN)]

            @plsc.parallel_loop(0, G * 8, unroll=8)
            def _(u):
                off = u * 16
                idx_r = rows_ch[slot, crow, pl.ds(off, 16)]
                idx_c = cols_ch[slot, crow, pl.ds(off, 16)]
                v = vals_ch[slot, crow, pl.ds(off, 16)]
                xg = plsc.load_gather(xp, [idx_c])
                plsc.addupdate_scatter(accp, [idx_r], v * xg)
                lg = plsc.load_gather(lamp, [idx_r])
                plsc.addupdate_scatter(accp, [idx_c + N], v * lg)

    cpx = pltpu.make_async_copy(x_hbm.at[pl.ds(c * 2 * N, 2 * N)], x2_v,
                                sem_x)
    cpl = pltpu.make_async_copy(lam_hbm.at[pl.ds(c * 2 * M, 2 * M)], lam2_v,
                                sem_l)
    cpx.start()
    cpl.start()
    for ch in range(NBUF - 1):
        start(jnp.int32(ch), ch)

    @plsc.parallel_loop(0, 2 * MN // 16, unroll=8)
    def _(k):
        acc_v[pl.ds(k * 16, 16)] = zero16

    cpx.wait()
    cpl.wait()

    def pipe(k, carry):
        for b in range(NBUF):
            ch = k * NBUF + b

            @pl.when(ch < NCH)
            def _():
                start(ch + (NBUF - 1), (b + NBUF - 1) % NBUF)
                wait(b)
                sanitize(ch, b)
                process_chunk(b)

        return carry

    lax.fori_loop(0, (NCH + NBUF - 1) // NBUF, pipe, 0)

    pltpu.sync_copy(acc_v, acc_sh.at[pl.ds(s * 2 * MN, 2 * MN)])
    plsc.subcore_barrier()

    pair = s // 8
    j = s - pair * 8
    rowid = c * 16 + s
    sl_off = pair * MN + j * 1024
    for t in range(16):
        pltpu.make_async_copy(acc_sh.at[pl.ds(t * 2 * MN + sl_off, 1024)],
                              tmp_v.at[pl.ds(t * 1024, 1024)],
                              sem_x).start()
    for t in range(16):
        pltpu.make_async_copy(acc_sh.at[pl.ds(t * 2 * MN + sl_off, 1024)],
                              tmp_v.at[pl.ds(t * 1024, 1024)],
                              sem_x).wait()

    def red16(o):
        a16 = tmp_v[pl.ds(o, 16)]
        for t in range(1, 16):
            a16 = a16 + tmp_v[pl.ds(t * 1024 + o, 16)]
        return a16

    @pl.when(j < 4)
    def _():
        pltpu.sync_copy(b_hbm.at[pl.ds(0, 4), pl.ds(j * 1024, 1024)], bc_v)

        def sbody(k, carry):
            sp, sd, sm = carry
            o = k * 16
            a16 = red16(o)
            bb = bc_v[2 * c + pair, pl.ds(o, 16)]
            ll = lam2_v[pl.ds(pair * M + j * 1024 + o, 16)]
            axmb = a16 - bb
            rp = jnp.maximum(axmb, 0.0)
            rn = jnp.maximum(-ll, 0.0)
            cm = ll * axmb
            return (sp + rp * rp, sd + rn * rn, sm + cm * cm)

        sp, sd, sm = lax.fori_loop(0, 64, sbody, (zero16, zero16, zero16))
        val = (W_PRIMAL * jnp.sum(sp) + W_DUAL * jnp.sum(sd)
               + W_COMP * jnp.sum(sm)) * INV_MB
        out_v[...] = jnp.full((16,), 1.0, jnp.float32) * val
        pltpu.sync_copy(out_v, out_hbm.at[pl.ds(rowid * 16, 16)])

    @pl.when(j >= 4)
    def _():
        pltpu.sync_copy(c_hbm.at[pl.ds(0, 4), pl.ds((j - 4) * 1024, 1024)],
                        bc_v)

        def sbody(k, st):
            o = k * 16
            a16 = red16(o)
            cc = bc_v[2 * c + pair, pl.ds(o, 16)]
            r = a16 + cc
            return st + r * r

        st = lax.fori_loop(0, 64, sbody, zero16)
        val = W_STAT * jnp.sum(st) * INV_MB
        out_v[...] = jnp.full((16,), 1.0, jnp.float32) * val
        pltpu.sync_copy(out_v, out_hbm.at[pl.ds(rowid * 16, 16)])


@jax.jit
def _run(x_hat, lam_hat, A_vals, A_rows, A_cols, b_pad, c_pad):
    mesh = plsc.VectorSubcoreMesh(core_axis_name="c", subcore_axis_name="s")
    kfn = pl.kernel(
        _sc_body,
        out_type=jax.ShapeDtypeStruct((32 * 16,), jnp.float32),
        mesh=mesh,
        compiler_params=pltpu.CompilerParams(needs_layout_passes=False,
                                             disable_bounds_checks=True),
        scratch_types=[
            pltpu.VMEM((NBUF, 4, CW), jnp.float32),
            pltpu.VMEM((NBUF, 4, CW), jnp.int32),
            pltpu.VMEM((NBUF, 4, CW), jnp.int32),
            pltpu.VMEM((2 * N,), jnp.float32),
            pltpu.VMEM((2 * M,), jnp.float32),
            pltpu.VMEM((2 * MN,), jnp.float32),
            pltpu.VMEM((16 * 1024,), jnp.float32),
            pltpu.VMEM((4, 1024), jnp.float32),
            pltpu.VMEM((16,), jnp.float32),
            pltpu.VMEM_SHARED((16 * 2 * MN,), jnp.float32),
            pltpu.SemaphoreType.DMA((NBUF,)),
            pltpu.SemaphoreType.DMA((NBUF,)),
            pltpu.SemaphoreType.DMA((NBUF,)),
            pltpu.SemaphoreType.DMA,
            pltpu.SemaphoreType.DMA,
        ],
    )
    out = kfn(x_hat, lam_hat, A_vals, A_rows, A_cols, b_pad, c_pad)
    return jnp.sum(out.reshape(32, 16)[:, 0])


def kernel(x_hat, lam_hat, A_vals, A_rows, A_cols, b_pad, c_pad):
    return _run(x_hat, lam_hat, A_vals, A_rows, A_cols, b_pad, c_pad)


# speedup vs baseline: 1.0409x; 1.0409x over previous
import jax
import jax.numpy as jnp
from jax import lax
from jax.experimental import pallas as pl
from jax.experimental.pallas import tpu as pltpu
from jax.experimental.pallas import tpu_sc as plsc

B, M, N = 4, 4096, 4096
NNZ = 167772
NBLK = (NNZ + 127) // 128
LAST_VALID = NNZ - (NBLK - 1) * 128
BPT = 82
G = 6
NCH = (BPT + G - 1) // G
NBUF = 4
CW = G * 128
MN = M + N
W_PRIMAL, W_DUAL, W_STAT, W_COMP = 0.1, 0.1, 0.6, 0.2
INV_MB = 1.0 / float(M * B)


def _sc_body(x_hbm, lam_hbm, vals_hbm, rows_hbm, cols_hbm, b_hbm, c_hbm,
             out_hbm,
             vals_ch, rows_ch, cols_ch, x2_v, lam2_v, acc_v, tmp_v, bc_v,
             out_v,
             acc_sh,
             sem_v, sem_r, sem_c, sem_x, sem_l):
    c = lax.axis_index("c")
    s = lax.axis_index("s")
    base = jnp.minimum(s * BPT, NBLK - G * NCH)
    lanes = lax.iota(jnp.int32, 16)
    zero16 = jnp.zeros((16,), jnp.float32)
    zero16i = jnp.zeros((16,), jnp.int32)

    def start(ch, slot):
        @pl.when(ch < NCH)
        def _():
            col0 = (base + ch * G) * 128
            for hbm, buf, sem in ((vals_hbm, vals_ch, sem_v),
                                  (rows_hbm, rows_ch, sem_r),
                                  (cols_hbm, cols_ch, sem_c)):
                pltpu.make_async_copy(
                    hbm.at[pl.ds(0, 4), pl.ds(col0, CW)],
                    buf.at[slot], sem.at[slot]).start()

    def wait(slot):
        for hbm, buf, sem in ((vals_hbm, vals_ch, sem_v),
                              (rows_hbm, rows_ch, sem_r),
                              (cols_hbm, cols_ch, sem_c)):
            pltpu.make_async_copy(
                hbm.at[pl.ds(0, 4), pl.ds(0, CW)],
                buf.at[slot], sem.at[slot]).wait()

    def sanitize(ch, slot):
        @pl.when(jnp.logical_and(s == 15, ch == 0))
        def _():
            nover = 16 * BPT - NBLK + (G * NCH - BPT)
            for r in range(4):
                for o in range(nover * 8):
                    vals_ch[slot, r, pl.ds(o * 16, 16)] = zero16

        @pl.when(jnp.logical_and(s < 15, ch == NCH - 1))
        def _():
            for r in range(4):
                for o in range((BPT - G * (NCH - 1)) * 8, G * 8):
                    vals_ch[slot, r, pl.ds(o * 16, 16)] = zero16

        @pl.when(jnp.logical_and(s == 15, ch == NCH - 1))
        def _():
            fo = (G - 1) * 128 + (LAST_VALID // 16) * 16
            keep = lanes < (LAST_VALID - (LAST_VALID // 16) * 16)
            for r in range(4):
                vals_ch[slot, r, pl.ds(fo, 16)] = jnp.where(
                    keep, vals_ch[slot, r, pl.ds(fo, 16)], 0.0)
                rows_ch[slot, r, pl.ds(fo, 16)] = jnp.where(
                    keep, rows_ch[slot, r, pl.ds(fo, 16)], 0)
                cols_ch[slot, r, pl.ds(fo, 16)] = jnp.where(
                    keep, cols_ch[slot, r, pl.ds(fo, 16)], 0)
                for o in range(fo + 16, CW, 16):
                    vals_ch[slot, r, pl.ds(o, 16)] = zero16
                    rows_ch[slot, r, pl.ds(o, 16)] = zero16i
                    cols_ch[slot, r, pl.ds(o, 16)] = zero16i

    def process_chunk(slot):
        for pair in (0, 1):
            crow = 2 * c + pair
            xp = x2_v.at[pl.ds(pair * N, N)]
            lamp = lam2_v.at[pl.ds(pair * M, M)]
            accp = acc_v.at[pl.ds(pair * MN, MN)]

            @plsc.parallel_loop(0, G * 8, unroll=8)
            def _(u):
                off = u * 16
                idx_r = rows_ch[slot, crow, pl.ds(off, 16)]
                idx_c = cols_ch[slot, crow, pl.ds(off, 16)]
                v = vals_ch[slot, crow, pl.ds(off, 16)]
                xg = plsc.load_gather(xp, [idx_c])
                plsc.addupdate_scatter(accp, [idx_r], v * xg)
                lg = plsc.load_gather(lamp, [idx_r])
                plsc.addupdate_scatter(accp, [idx_c + N], v * lg)

    cpx = pltpu.make_async_copy(x_hbm.at[pl.ds(c * 2 * N, 2 * N)], x2_v,
                                sem_x)
    cpl = pltpu.make_async_copy(lam_hbm.at[pl.ds(c * 2 * M, 2 * M)], lam2_v,
                                sem_l)
    cpx.start()
    cpl.start()
    for ch in range(NBUF - 1):
        start(jnp.int32(ch), ch)

    @plsc.parallel_loop(0, 2 * MN // 16, unroll=8)
    def _(k):
        acc_v[pl.ds(k * 16, 16)] = zero16

    cpx.wait()
    cpl.wait()

    def pipe(k, carry):
        for b in range(NBUF):
            ch = k * NBUF + b

            @pl.when(ch < NCH)
            def _():
                start(ch + (NBUF - 1), (b + NBUF - 1) % NBUF)
                wait(b)
                sanitize(ch, b)
                process_chunk(b)

        return carry

    lax.fori_loop(0, (NCH + NBUF - 1) // NBUF, pipe, 0)

    pltpu.sync_copy(acc_v, acc_sh.at[pl.ds(s * 2 * MN, 2 * MN)])
    plsc.subcore_barrier()

    pair = s // 8
    j = s - pair * 8
    rowid = c * 16 + s
    sl_off = pair * MN + j * 1024
    for t in range(16):
        pltpu.make_async_copy(acc_sh.at[pl.ds(t * 2 * MN + sl_off, 1024)],
                              tmp_v.at[pl.ds(t * 1024, 1024)],
                              sem_x).start()
    for t in range(16):
        pltpu.make_async_copy(acc_sh.at[pl.ds(t * 2 * MN + sl_off, 1024)],
                              tmp_v.at[pl.ds(t * 1024, 1024)],
                              sem_x).wait()

    def red16(o):
        a16 = tmp_v[pl.ds(o, 16)]
        for t in range(1, 16):
            a16 = a16 + tmp_v[pl.ds(t * 1024 + o, 16)]
        return a16

    @pl.when(j < 4)
    def _():
        pltpu.sync_copy(b_hbm.at[pl.ds(0, 4), pl.ds(j * 1024, 1024)], bc_v)

        def sbody(k, carry):
            sp, sd, sm = carry
            o = k * 16
            a16 = red16(o)
            bb = bc_v[2 * c + pair, pl.ds(o, 16)]
            ll = lam2_v[pl.ds(pair * M + j * 1024 + o, 16)]
            axmb = a16 - bb
            rp = jnp.maximum(axmb, 0.0)
            rn = jnp.maximum(-ll, 0.0)
            cm = ll * axmb
            return (sp + rp * rp, sd + rn * rn, sm + cm * cm)

        sp, sd, sm = lax.fori_loop(0, 64, sbody, (zero16, zero16, zero16))
        val = (W_PRIMAL * jnp.sum(sp) + W_DUAL * jnp.sum(sd)
               + W_COMP * jnp.sum(sm)) * INV_MB
        out_v[...] = jnp.full((16,), 1.0, jnp.float32) * val
        pltpu.sync_copy(out_v, out_hbm.at[pl.ds(rowid * 16, 16)])

    @pl.when(j >= 4)
    def _():
        pltpu.sync_copy(c_hbm.at[pl.ds(0, 4), pl.ds((j - 4) * 1024, 1024)],
                        bc_v)

        def sbody(k, st):
            o = k * 16
            a16 = red16(o)
            cc = bc_v[2 * c + pair, pl.ds(o, 16)]
            r = a16 + cc
            return st + r * r

        st = lax.fori_loop(0, 64, sbody, zero16)
        val = W_STAT * jnp.sum(st) * INV_MB
        out_v[...] = jnp.full((16,), 1.0, jnp.float32) * val
        pltpu.sync_copy(out_v, out_hbm.at[pl.ds(rowid * 16, 16)])


@jax.jit
def _run(x_hat, lam_hat, A_vals, A_rows, A_cols, b_pad, c_pad):
    mesh = plsc.VectorSubcoreMesh(core_axis_name="c", subcore_axis_name="s")
    kfn = pl.kernel(
        _sc_body,
        out_type=jax.ShapeDtypeStruct((32 * 16,), jnp.float32),
        mesh=mesh,
        compiler_params=pltpu.CompilerParams(needs_layout_passes=False,
                                             disable_bounds_checks=True),
        scratch_types=[
            pltpu.VMEM((NBUF, 4, CW), jnp.float32),
            pltpu.VMEM((NBUF, 4, CW), jnp.int32),
            pltpu.VMEM((NBUF, 4, CW), jnp.int32),
            pltpu.VMEM((2 * N,), jnp.float32),
            pltpu.VMEM((2 * M,), jnp.float32),
            pltpu.VMEM((2 * MN,), jnp.float32),
            pltpu.VMEM((16 * 1024,), jnp.float32),
            pltpu.VMEM((4, 1024), jnp.float32),
            pltpu.VMEM((16,), jnp.float32),
            pltpu.VMEM_SHARED((16 * 2 * MN,), jnp.float32),
            pltpu.SemaphoreType.DMA((NBUF,)),
            pltpu.SemaphoreType.DMA((NBUF,)),
            pltpu.SemaphoreType.DMA((NBUF,)),
            pltpu.SemaphoreType.DMA,
            pltpu.SemaphoreType.DMA,
        ],
    )
    out = kfn(x_hat, lam_hat, A_vals, A_rows, A_cols, b_pad, c_pad)
    return jnp.sum(out.reshape(32, 16)[:, 0])


def kernel(x_hat, lam_hat, A_vals, A_rows, A_cols, b_pad, c_pad):
    return _run(x_hat, lam_hat, A_vals, A_rows, A_cols, b_pad, c_pad)


# speedup vs baseline: 1.0427x; 1.0017x over previous
import jax
import jax.numpy as jnp
from jax import lax
from jax.experimental import pallas as pl
from jax.experimental.pallas import tpu as pltpu
from jax.experimental.pallas import tpu_sc as plsc

B, M, N = 4, 4096, 4096
NNZ = 167772
NBLK = (NNZ + 127) // 128
LAST_VALID = NNZ - (NBLK - 1) * 128
BPT = 82
G = 4
NCH = (BPT + G - 1) // G
NBUF = 4
CW = G * 128
MN = M + N
W_PRIMAL, W_DUAL, W_STAT, W_COMP = 0.1, 0.1, 0.6, 0.2
INV_MB = 1.0 / float(M * B)


def _sc_body(x_hbm, lam_hbm, vals_hbm, rows_hbm, cols_hbm, b_hbm, c_hbm,
             out_hbm,
             vals_ch, rows_ch, cols_ch, x2_v, lam2_v, acc_v, tmp_v, bc_v,
             out_v,
             acc_sh,
             sem_v, sem_r, sem_c, sem_x, sem_l):
    c = lax.axis_index("c")
    s = lax.axis_index("s")
    base = jnp.minimum(s * BPT, NBLK - G * NCH)
    lanes = lax.iota(jnp.int32, 16)
    zero16 = jnp.zeros((16,), jnp.float32)
    zero16i = jnp.zeros((16,), jnp.int32)

    def start(ch, slot):
        @pl.when(ch < NCH)
        def _():
            col0 = (base + ch * G) * 128
            for hbm, buf, sem in ((vals_hbm, vals_ch, sem_v),
                                  (rows_hbm, rows_ch, sem_r),
                                  (cols_hbm, cols_ch, sem_c)):
                pltpu.make_async_copy(
                    hbm.at[pl.ds(0, 4), pl.ds(col0, CW)],
                    buf.at[slot], sem.at[slot]).start()

    def wait(slot):
        for hbm, buf, sem in ((vals_hbm, vals_ch, sem_v),
                              (rows_hbm, rows_ch, sem_r),
                              (cols_hbm, cols_ch, sem_c)):
            pltpu.make_async_copy(
                hbm.at[pl.ds(0, 4), pl.ds(0, CW)],
                buf.at[slot], sem.at[slot]).wait()

    def sanitize(ch, slot):
        @pl.when(jnp.logical_and(s == 15, ch == 0))
        def _():
            nover = 16 * BPT - NBLK + (G * NCH - BPT)
            for r in range(4):
                for o in range(nover * 8):
                    vals_ch[slot, r, pl.ds(o * 16, 16)] = zero16

        @pl.when(jnp.logical_and(s < 15, ch == NCH - 1))
        def _():
            for r in range(4):
                for o in range((BPT - G * (NCH - 1)) * 8, G * 8):
                    vals_ch[slot, r, pl.ds(o * 16, 16)] = zero16

        @pl.when(jnp.logical_and(s == 15, ch == NCH - 1))
        def _():
            fo = (G - 1) * 128 + (LAST_VALID // 16) * 16
            keep = lanes < (LAST_VALID - (LAST_VALID // 16) * 16)
            for r in range(4):
                vals_ch[slot, r, pl.ds(fo, 16)] = jnp.where(
                    keep, vals_ch[slot, r, pl.ds(fo, 16)], 0.0)
                rows_ch[slot, r, pl.ds(fo, 16)] = jnp.where(
                    keep, rows_ch[slot, r, pl.ds(fo, 16)], 0)
                cols_ch[slot, r, pl.ds(fo, 16)] = jnp.where(
                    keep, cols_ch[slot, r, pl.ds(fo, 16)], 0)
                for o in range(fo + 16, CW, 16):
                    vals_ch[slot, r, pl.ds(o, 16)] = zero16
                    rows_ch[slot, r, pl.ds(o, 16)] = zero16i
                    cols_ch[slot, r, pl.ds(o, 16)] = zero16i

    def process_chunk(slot):
        for pair in (0, 1):
            crow = 2 * c + pair
            xp = x2_v.at[pl.ds(pair * N, N)]
            lamp = lam2_v.at[pl.ds(pair * M, M)]
            accp = acc_v.at[pl.ds(pair * MN, MN)]

            @plsc.parallel_loop(0, G * 8, unroll=8)
            def _(u):
                off = u * 16
                idx_r = rows_ch[slot, crow, pl.ds(off, 16)]
                idx_c = cols_ch[slot, crow, pl.ds(off, 16)]
                v = vals_ch[slot, crow, pl.ds(off, 16)]
                xg = plsc.load_gather(xp, [idx_c])
                plsc.addupdate_scatter(accp, [idx_r], v * xg)
                lg = plsc.load_gather(lamp, [idx_r])
                plsc.addupdate_scatter(accp, [idx_c + N], v * lg)

    cpx = pltpu.make_async_copy(x_hbm.at[pl.ds(c * 2 * N, 2 * N)], x2_v,
                                sem_x)
    cpl = pltpu.make_async_copy(lam_hbm.at[pl.ds(c * 2 * M, 2 * M)], lam2_v,
                                sem_l)
    cpx.start()
    cpl.start()
    for ch in range(NBUF - 1):
        start(jnp.int32(ch), ch)

    @plsc.parallel_loop(0, 2 * MN // 16, unroll=8)
    def _(k):
        acc_v[pl.ds(k * 16, 16)] = zero16

    cpx.wait()
    cpl.wait()

    def pipe(k, carry):
        for b in range(NBUF):
            ch = k * NBUF + b

            @pl.when(ch < NCH)
            def _():
                start(ch + (NBUF - 1), (b + NBUF - 1) % NBUF)
                wait(b)
                sanitize(ch, b)
                process_chunk(b)

        return carry

    lax.fori_loop(0, (NCH + NBUF - 1) // NBUF, pipe, 0)

    pltpu.sync_copy(acc_v, acc_sh.at[pl.ds(s * 2 * MN, 2 * MN)])
    plsc.subcore_barrier()

    pair = s // 8
    j = s - pair * 8
    rowid = c * 16 + s
    sl_off = pair * MN + j * 1024
    for t in range(16):
        pltpu.make_async_copy(acc_sh.at[pl.ds(t * 2 * MN + sl_off, 1024)],
                              tmp_v.at[pl.ds(t * 1024, 1024)],
                              sem_x).start()
    for t in range(16):
        pltpu.make_async_copy(acc_sh.at[pl.ds(t * 2 * MN + sl_off, 1024)],
                              tmp_v.at[pl.ds(t * 1024, 1024)],
                              sem_x).wait()

    def red16(o):
        a16 = tmp_v[pl.ds(o, 16)]
        for t in range(1, 16):
            a16 = a16 + tmp_v[pl.ds(t * 1024 + o, 16)]
        return a16

    @pl.when(j < 4)
    def _():
        pltpu.sync_copy(b_hbm.at[pl.ds(0, 4), pl.ds(j * 1024, 1024)], bc_v)

        def sbody(k, carry):
            sp, sd, sm = carry
            o = k * 16
            a16 = red16(o)
            bb = bc_v[2 * c + pair, pl.ds(o, 16)]
            ll = lam2_v[pl.ds(pair * M + j * 1024 + o, 16)]
            axmb = a16 - bb
            rp = jnp.maximum(axmb, 0.0)
            rn = jnp.maximum(-ll, 0.0)
            cm = ll * axmb
            return (sp + rp * rp, sd + rn * rn, sm + cm * cm)

        sp, sd, sm = lax.fori_loop(0, 64, sbody, (zero16, zero16, zero16))
        val = (W_PRIMAL * jnp.sum(sp) + W_DUAL * jnp.sum(sd)
               + W_COMP * jnp.sum(sm)) * INV_MB
        out_v[...] = jnp.full((16,), 1.0, jnp.float32) * val
        pltpu.sync_copy(out_v, out_hbm.at[pl.ds(rowid * 16, 16)])

    @pl.when(j >= 4)
    def _():
        pltpu.sync_copy(c_hbm.at[pl.ds(0, 4), pl.ds((j - 4) * 1024, 1024)],
                        bc_v)

        def sbody(k, st):
            o = k * 16
            a16 = red16(o)
            cc = bc_v[2 * c + pair, pl.ds(o, 16)]
            r = a16 + cc
            return st + r * r

        st = lax.fori_loop(0, 64, sbody, zero16)
        val = W_STAT * jnp.sum(st) * INV_MB
        out_v[...] = jnp.full((16,), 1.0, jnp.float32) * val
        pltpu.sync_copy(out_v, out_hbm.at[pl.ds(rowid * 16, 16)])


@jax.jit
def _run(x_hat, lam_hat, A_vals, A_rows, A_cols, b_pad, c_pad):
    mesh = plsc.VectorSubcoreMesh(core_axis_name="c", subcore_axis_name="s")
    kfn = pl.kernel(
        _sc_body,
        out_type=jax.ShapeDtypeStruct((32 * 16,), jnp.float32),
        mesh=mesh,
        compiler_params=pltpu.CompilerParams(needs_layout_passes=False,
                                             disable_bounds_checks=True),
        scratch_types=[
            pltpu.VMEM((NBUF, 4, CW), jnp.float32),
            pltpu.VMEM((NBUF, 4, CW), jnp.int32),
            pltpu.VMEM((NBUF, 4, CW), jnp.int32),
            pltpu.VMEM((2 * N,), jnp.float32),
            pltpu.VMEM((2 * M,), jnp.float32),
            pltpu.VMEM((2 * MN,), jnp.float32),
            pltpu.VMEM((16 * 1024,), jnp.float32),
            pltpu.VMEM((4, 1024), jnp.float32),
            pltpu.VMEM((16,), jnp.float32),
            pltpu.VMEM_SHARED((16 * 2 * MN,), jnp.float32),
            pltpu.SemaphoreType.DMA((NBUF,)),
            pltpu.SemaphoreType.DMA((NBUF,)),
            pltpu.SemaphoreType.DMA((NBUF,)),
            pltpu.SemaphoreType.DMA,
            pltpu.SemaphoreType.DMA,
        ],
    )
    out = kfn(x_hat, lam_hat, A_vals, A_rows, A_cols, b_pad, c_pad)
    return jnp.sum(out.reshape(32, 16)[:, 0])


def kernel(x_hat, lam_hat, A_vals, A_rows, A_cols, b_pad, c_pad):
    return _run(x_hat, lam_hat, A_vals, A_rows, A_cols, b_pad, c_pad)


# speedup vs baseline: 1.0479x; 1.0050x over previous
import jax
import jax.numpy as jnp
from jax import lax
from jax.experimental import pallas as pl
from jax.experimental.pallas import tpu as pltpu
from jax.experimental.pallas import tpu_sc as plsc

B, M, N = 4, 4096, 4096
NNZ = 167772
NBLK = (NNZ + 127) // 128
LAST_VALID = NNZ - (NBLK - 1) * 128
BPT = 82
G = 4
NCH = (BPT + G - 1) // G
NBUF = 4
CW = G * 128
MN = M + N
W_PRIMAL, W_DUAL, W_STAT, W_COMP = 0.1, 0.1, 0.6, 0.2
INV_MB = 1.0 / float(M * B)


def _sc_body(x_hbm, lam_hbm, vals_hbm, rows_hbm, cols_hbm, b_hbm, c_hbm,
             out_hbm,
             vals_ch, rows_ch, cols_ch, x2_v, lam2_v, acc_v, tmp_v, bc_v,
             out_v,
             acc_sh,
             sem_v, sem_r, sem_c, sem_x, sem_l):
    c = lax.axis_index("c")
    s = lax.axis_index("s")
    base = jnp.minimum(s * BPT, NBLK - G * NCH)
    lanes = lax.iota(jnp.int32, 16)
    zero16 = jnp.zeros((16,), jnp.float32)
    zero16i = jnp.zeros((16,), jnp.int32)

    def start(ch, slot):
        @pl.when(ch < NCH)
        def _():
            col0 = (base + ch * G) * 128
            for hbm, buf, sem in ((vals_hbm, vals_ch, sem_v),
                                  (rows_hbm, rows_ch, sem_r),
                                  (cols_hbm, cols_ch, sem_c)):
                pltpu.make_async_copy(
                    hbm.at[pl.ds(0, 4), pl.ds(col0, CW)],
                    buf.at[slot], sem.at[slot]).start()

    def wait(slot):
        for hbm, buf, sem in ((vals_hbm, vals_ch, sem_v),
                              (rows_hbm, rows_ch, sem_r),
                              (cols_hbm, cols_ch, sem_c)):
            pltpu.make_async_copy(
                hbm.at[pl.ds(0, 4), pl.ds(0, CW)],
                buf.at[slot], sem.at[slot]).wait()

    def sanitize(ch, slot):
        @pl.when(jnp.logical_and(s == 15, ch == 0))
        def _():
            nover = 16 * BPT - NBLK + (G * NCH - BPT)
            for r in range(4):
                for o in range(nover * 8):
                    vals_ch[slot, r, pl.ds(o * 16, 16)] = zero16

        @pl.when(jnp.logical_and(s < 15, ch == NCH - 1))
        def _():
            for r in range(4):
                for o in range((BPT - G * (NCH - 1)) * 8, G * 8):
                    vals_ch[slot, r, pl.ds(o * 16, 16)] = zero16

        @pl.when(jnp.logical_and(s == 15, ch == NCH - 1))
        def _():
            fo = (G - 1) * 128 + (LAST_VALID // 16) * 16
            keep = lanes < (LAST_VALID - (LAST_VALID // 16) * 16)
            for r in range(4):
                vals_ch[slot, r, pl.ds(fo, 16)] = jnp.where(
                    keep, vals_ch[slot, r, pl.ds(fo, 16)], 0.0)
                rows_ch[slot, r, pl.ds(fo, 16)] = jnp.where(
                    keep, rows_ch[slot, r, pl.ds(fo, 16)], 0)
                cols_ch[slot, r, pl.ds(fo, 16)] = jnp.where(
                    keep, cols_ch[slot, r, pl.ds(fo, 16)], 0)
                for o in range(fo + 16, CW, 16):
                    vals_ch[slot, r, pl.ds(o, 16)] = zero16
                    rows_ch[slot, r, pl.ds(o, 16)] = zero16i
                    cols_ch[slot, r, pl.ds(o, 16)] = zero16i

    def process_chunk(slot):
        for pair in (0, 1):
            crow = 2 * c + pair
            xp = x2_v.at[pl.ds(pair * N, N)]
            lamp = lam2_v.at[pl.ds(pair * M, M)]
            accp = acc_v.at[pl.ds(pair * MN, MN)]

            @plsc.parallel_loop(0, G * 8, unroll=16)
            def _(u):
                off = u * 16
                idx_r = rows_ch[slot, crow, pl.ds(off, 16)]
                idx_c = cols_ch[slot, crow, pl.ds(off, 16)]
                v = vals_ch[slot, crow, pl.ds(off, 16)]
                xg = plsc.load_gather(xp, [idx_c])
                plsc.addupdate_scatter(accp, [idx_r], v * xg)
                lg = plsc.load_gather(lamp, [idx_r])
                plsc.addupdate_scatter(accp, [idx_c + N], v * lg)

    cpx = pltpu.make_async_copy(x_hbm.at[pl.ds(c * 2 * N, 2 * N)], x2_v,
                                sem_x)
    cpl = pltpu.make_async_copy(lam_hbm.at[pl.ds(c * 2 * M, 2 * M)], lam2_v,
                                sem_l)
    cpx.start()
    cpl.start()
    for ch in range(NBUF - 1):
        start(jnp.int32(ch), ch)

    @plsc.parallel_loop(0, 2 * MN // 16, unroll=8)
    def _(k):
        acc_v[pl.ds(k * 16, 16)] = zero16

    cpx.wait()
    cpl.wait()

    def pipe(k, carry):
        for b in range(NBUF):
            ch = k * NBUF + b

            @pl.when(ch < NCH)
            def _():
                start(ch + (NBUF - 1), (b + NBUF - 1) % NBUF)
                wait(b)
                sanitize(ch, b)
                process_chunk(b)

        return carry

    lax.fori_loop(0, (NCH + NBUF - 1) // NBUF, pipe, 0)

    pltpu.sync_copy(acc_v, acc_sh.at[pl.ds(s * 2 * MN, 2 * MN)])
    plsc.subcore_barrier()

    pair = s // 8
    j = s - pair * 8
    rowid = c * 16 + s
    sl_off = pair * MN + j * 1024
    for t in range(16):
        pltpu.make_async_copy(acc_sh.at[pl.ds(t * 2 * MN + sl_off, 1024)],
                              tmp_v.at[pl.ds(t * 1024, 1024)],
                              sem_x).start()
    for t in range(16):
        pltpu.make_async_copy(acc_sh.at[pl.ds(t * 2 * MN + sl_off, 1024)],
                              tmp_v.at[pl.ds(t * 1024, 1024)],
                              sem_x).wait()

    def red16(o):
        a16 = tmp_v[pl.ds(o, 16)]
        for t in range(1, 16):
            a16 = a16 + tmp_v[pl.ds(t * 1024 + o, 16)]
        return a16

    @pl.when(j < 4)
    def _():
        pltpu.sync_copy(b_hbm.at[pl.ds(0, 4), pl.ds(j * 1024, 1024)], bc_v)

        def sbody(k, carry):
            sp, sd, sm = carry
            o = k * 16
            a16 = red16(o)
            bb = bc_v[2 * c + pair, pl.ds(o, 16)]
            ll = lam2_v[pl.ds(pair * M + j * 1024 + o, 16)]
            axmb = a16 - bb
            rp = jnp.maximum(axmb, 0.0)
            rn = jnp.maximum(-ll, 0.0)
            cm = ll * axmb
            return (sp + rp * rp, sd + rn * rn, sm + cm * cm)

        sp, sd, sm = lax.fori_loop(0, 64, sbody, (zero16, zero16, zero16))
        val = (W_PRIMAL * jnp.sum(sp) + W_DUAL * jnp.sum(sd)
               + W_COMP * jnp.sum(sm)) * INV_MB
        out_v[...] = jnp.full((16,), 1.0, jnp.float32) * val
        pltpu.sync_copy(out_v, out_hbm.at[pl.ds(rowid * 16, 16)])

    @pl.when(j >= 4)
    def _():
        pltpu.sync_copy(c_hbm.at[pl.ds(0, 4), pl.ds((j - 4) * 1024, 1024)],
                        bc_v)

        def sbody(k, st):
            o = k * 16
            a16 = red16(o)
            cc = bc_v[2 * c + pair, pl.ds(o, 16)]
            r = a16 + cc
            return st + r * r

        st = lax.fori_loop(0, 64, sbody, zero16)
        val = W_STAT * jnp.sum(st) * INV_MB
        out_v[...] = jnp.full((16,), 1.0, jnp.float32) * val
        pltpu.sync_copy(out_v, out_hbm.at[pl.ds(rowid * 16, 16)])


@jax.jit
def _run(x_hat, lam_hat, A_vals, A_rows, A_cols, b_pad, c_pad):
    mesh = plsc.VectorSubcoreMesh(core_axis_name="c", subcore_axis_name="s")
    kfn = pl.kernel(
        _sc_body,
        out_type=jax.ShapeDtypeStruct((32 * 16,), jnp.float32),
        mesh=mesh,
        compiler_params=pltpu.CompilerParams(needs_layout_passes=False,
                                             disable_bounds_checks=True),
        scratch_types=[
            pltpu.VMEM((NBUF, 4, CW), jnp.float32),
            pltpu.VMEM((NBUF, 4, CW), jnp.int32),
            pltpu.VMEM((NBUF, 4, CW), jnp.int32),
            pltpu.VMEM((2 * N,), jnp.float32),
            pltpu.VMEM((2 * M,), jnp.float32),
            pltpu.VMEM((2 * MN,), jnp.float32),
            pltpu.VMEM((16 * 1024,), jnp.float32),
            pltpu.VMEM((4, 1024), jnp.float32),
            pltpu.VMEM((16,), jnp.float32),
            pltpu.VMEM_SHARED((16 * 2 * MN,), jnp.float32),
            pltpu.SemaphoreType.DMA((NBUF,)),
            pltpu.SemaphoreType.DMA((NBUF,)),
            pltpu.SemaphoreType.DMA((NBUF,)),
            pltpu.SemaphoreType.DMA,
            pltpu.SemaphoreType.DMA,
        ],
    )
    out = kfn(x_hat, lam_hat, A_vals, A_rows, A_cols, b_pad, c_pad)
    return jnp.sum(out.reshape(32, 16)[:, 0])


def kernel(x_hat, lam_hat, A_vals, A_rows, A_cols, b_pad, c_pad):
    return _run(x_hat, lam_hat, A_vals, A_rows, A_cols, b_pad, c_pad)
